# baseline (speedup 1.0000x reference)
import functools

import jax
import jax.numpy as jnp
from jax import lax
from jax.experimental import pallas as pl
from jax.experimental.pallas import tpu as pltpu
from jax.experimental.pallas import tpu_sc as plsc

NV = 10000
NE = 5000
NEP = 5008
NNZ = 320000
CHUNK = 4000
NGRP = CHUNK // 16
NCHUNK = NNZ // CHUNK
NC = 2
NS = 16
NW = NC * NS


def _mm_body(w_ref, x_ref, b_ref, o_ref, *, dims):
    o_ref[...] = lax.dot_general(
        w_ref[...], x_ref[...], dims, preferred_element_type=jnp.float32
    ) + b_ref[...]


def _mm_xt(W, X, b):
    do = W.shape[1]
    return pl.pallas_call(
        functools.partial(_mm_body, dims=(((0,), (1,)), ((), ()))),
        out_shape=jax.ShapeDtypeStruct((do, X.shape[0]), jnp.float32),
    )(W, X, b.reshape(do, 1))


def _mm_tt(W, Zt, b):
    do = W.shape[1]
    return pl.pallas_call(
        functools.partial(_mm_body, dims=(((0,), (0,)), ((), ()))),
        out_shape=jax.ShapeDtypeStruct((do, Zt.shape[1]), jnp.float32),
    )(W, Zt, b.reshape(do, 1))


def _pack_ids_body(v_ref, e_ref, o_ref):
    o_ref[...] = jnp.bitwise_or(jnp.left_shift(e_ref[...], 14), v_ref[...])


def _pack_ids(v32, e32):
    return pl.pallas_call(
        _pack_ids_body,
        out_shape=jax.ShapeDtypeStruct(v32.shape, jnp.int32),
    )(v32, e32)


def _zero_fill(ref, n16):
    zeros16 = jnp.zeros((16,), jnp.float32)

    def body(j, _):
        ref[pl.ds(j * 16, 16)] = zeros16
        return 0

    lax.fori_loop(0, n16, body, 0, unroll=4)


def _sc_body(*refs, C, first, relu):
    if first:
        yt, pids, zt, rvd_out, red_out = refs[:5]
        rest = refs[5:]
    else:
        yt, pids, rvd_in, red_in, zt = refs[:5]
        rest = refs[5:]
    npair = C // 2
    ab = rest[:C]
    eb = rest[C:2 * C]
    pxb = rest[2 * C:2 * C + npair]
    peb = rest[2 * C + npair:2 * C + 2 * npair]
    vd, ed, pk0, pk1, sem0, sem1 = rest[2 * C + 2 * npair:]

    wid = lax.axis_index("s") * NC + lax.axis_index("c")
    row0 = wid * C

    ones16 = jnp.full((16,), 1.0, jnp.float32)

    for c in range(C):
        pltpu.sync_copy(yt.at[pl.ds((row0 + c) * NV, NV)], ab[c])

    def pkx(j, _):
        s = pl.ds(j * 16, 16)
        for p in range(npair):
            w = plsc.pack(ab[2 * p][s], ab[2 * p + 1][s],
                          format=plsc.PackFormat.INTERLEAVED)
            pxb[p][s] = plsc.bitcast(w, jnp.float32)
        return 0
    lax.fori_loop(0, NV // 16, pkx, 0, unroll=4)

    for c in range(C):
        _zero_fill(eb[c], NEP // 16)
    if first:
        _zero_fill(vd, NV // 16)
        _zero_fill(ed, NEP // 16)
    else:
        pltpu.sync_copy(rvd_in, vd)
        pltpu.sync_copy(red_in, ed)

    def _issue(k, pkb, sem):
        base = pl.multiple_of(jnp.minimum(k * CHUNK, NNZ - CHUNK), 8)
        pltpu.async_copy(pids.at[pl.ds(base, CHUNK)], pkb, sem)

    def _drain(pkb, sem):
        pltpu.make_async_copy(pids.at[pl.ds(0, CHUNK)], pkb, sem).wait()

    def _stream(proc_chunk):
        _issue(0, pk0, sem0)
        _issue(1, pk1, sem1)

        def pair(kk, _):
            k0 = 2 * kk
            _drain(pk0, sem0)
            proc_chunk(pk0)
            _issue(k0 + 2, pk0, sem0)
            _drain(pk1, sem1)
            proc_chunk(pk1)
            _issue(k0 + 3, pk1, sem1)
            return 0

        lax.fori_loop(0, NCHUNK // 2, pair, 0)
        _drain(pk0, sem0)
        _drain(pk1, sem1)

    def _decode(pk16):
        v16 = jnp.bitwise_and(pk16, 16383)
        e16 = lax.shift_right_logical(pk16, 14)
        return v16, e16

    def pass1_chunk(pk):
        v0, e0 = _decode(pk[pl.ds(0, 16)])
        w0 = [plsc.load_gather(pxb[p], [v0]) for p in range(npair)]

        def grp(g, carry):
            v16, e16 = carry[0], carry[1]
            ws = carry[2:]
            offn = pl.ds(jnp.minimum((g + 1) * 16, CHUNK - 16), 16)
            vn, en = _decode(pk[offn])
            wn = [plsc.load_gather(pxb[p], [vn]) for p in range(npair)]
            if first:
                plsc.addupdate_scatter(vd, [v16], ones16)
                plsc.addupdate_scatter(ed, [e16], ones16)
            for p in range(npair):
                a, b = plsc.unpack(plsc.bitcast(ws[p], jnp.bfloat16),
                                   format=plsc.PackFormat.INTERLEAVED)
                plsc.addupdate_scatter(eb[2 * p], [e16], a)
                plsc.addupdate_scatter(eb[2 * p + 1], [e16], b)
            return (vn, en, *wn)

        lax.fori_loop(0, NGRP, grp, (v0, e0, *w0), unroll=8)

    _stream(pass1_chunk)

    if first:
        def rvd_loop(j, _):
            s = pl.ds(j * 16, 16)
            vd[s] = 1.0 / jnp.maximum(vd[s], 1.0)
            return 0
        lax.fori_loop(0, NV // 16, rvd_loop, 0, unroll=4)

        def red_loop(j, _):
            s = pl.ds(j * 16, 16)
            ed[s] = 1.0 / jnp.maximum(ed[s], 1.0)
            return 0
        lax.fori_loop(0, NEP // 16, red_loop, 0, unroll=4)

    def esc(j, _):
        s = pl.ds(j * 16, 16)
        r = ed[s]
        for p in range(npair):
            a = eb[2 * p][s] * r
            b = eb[2 * p + 1][s] * r
            w = plsc.pack(a, b, format=plsc.PackFormat.INTERLEAVED)
            peb[p][s] = plsc.bitcast(w, jnp.float32)
        return 0
    lax.fori_loop(0, NEP // 16, esc, 0, unroll=4)

    for c in range(C):
        _zero_fill(ab[c], NV // 16)

    def pass2_chunk(pk):
        v0, e0 = _decode(pk[pl.ds(0, 16)])
        w0 = [plsc.load_gather(peb[p], [e0]) for p in range(npair)]

        def grp(g, carry):
            v16, e16 = carry[0], carry[1]
            ws = carry[2:]
            offn = pl.ds(jnp.minimum((g + 1) * 16, CHUNK - 16), 16)
            vn, en = _decode(pk[offn])
            wn = [plsc.load_gather(peb[p], [en]) for p in range(npair)]
            for p in range(npair):
                a, b = plsc.unpack(plsc.bitcast(ws[p], jnp.bfloat16),
                                   format=plsc.PackFormat.INTERLEAVED)
                plsc.addupdate_scatter(ab[2 * p], [v16], a)
                plsc.addupdate_scatter(ab[2 * p + 1], [v16], b)
            return (vn, en, *wn)

        lax.fori_loop(0, NGRP, grp, (v0, e0, *w0), unroll=8)

    _stream(pass2_chunk)

    def vsc(j, _):
        s = pl.ds(j * 16, 16)
        r = vd[s]
        for c in range(C):
            x = ab[c][s] * r
            if relu:
                x = jnp.maximum(x, 0.0)
            ab[c][s] = x
        return 0
    lax.fori_loop(0, NV // 16, vsc, 0, unroll=4)

    for c in range(C):
        pltpu.sync_copy(ab[c], zt.at[pl.ds((row0 + c) * NV, NV)])

    if first:
        @pl.when(wid == 0)
        def _():
            pltpu.sync_copy(vd, rvd_out)
            pltpu.sync_copy(ed, red_out)


def _make_sc(C, first, relu):
    d = C * NW
    out_type = [jax.ShapeDtypeStruct((d * NV,), jnp.float32)]
    if first:
        out_type += [jax.ShapeDtypeStruct((NV,), jnp.float32),
                     jax.ShapeDtypeStruct((NEP,), jnp.float32)]
    scratch = (
        [pltpu.VMEM((NV,), jnp.float32) for _ in range(C)]
        + [pltpu.VMEM((NEP,), jnp.float32) for _ in range(C)]
        + [pltpu.VMEM((NV,), jnp.float32) for _ in range(C // 2)]
        + [pltpu.VMEM((NEP,), jnp.float32) for _ in range(C // 2)]
        + [
            pltpu.VMEM((NV,), jnp.float32),
            pltpu.VMEM((NEP,), jnp.float32),
            pltpu.VMEM((CHUNK,), jnp.int32),
            pltpu.VMEM((CHUNK,), jnp.int32),
            pltpu.SemaphoreType.DMA,
            pltpu.SemaphoreType.DMA,
        ]
    )
    mesh = plsc.VectorSubcoreMesh(core_axis_name="c", subcore_axis_name="s")
    return pl.kernel(
        functools.partial(_sc_body, C=C, first=first, relu=relu),
        out_type=out_type,
        mesh=mesh,
        scratch_types=scratch,
        compiler_params=pltpu.CompilerParams(needs_layout_passes=False),
    )


def kernel(X, v_ids, e_ids, W0, b0, W1, b1, W2, b2):
    pk = _pack_ids(v_ids.astype(jnp.int32), e_ids.astype(jnp.int32))

    sc_first = _make_sc(4, True, True)
    sc_mid = _make_sc(4, False, True)
    sc_last = _make_sc(2, False, False)

    y0 = _mm_xt(W0, X, b0)
    z0f, rvd, red = sc_first(y0.reshape(-1), pk)
    z0 = z0f.reshape(128, NV)
    y1 = _mm_tt(W1, z0, b1)
    (z1f,) = sc_mid(y1.reshape(-1), pk, rvd, red)
    z1 = z1f.reshape(128, NV)
    y2 = _mm_tt(W2, z1, b2)
    (z2f,) = sc_last(y2.reshape(-1), pk, rvd, red)
    return z2f.reshape(64, NV).T


# speedup vs baseline: 1.0030x; 1.0030x over previous
import functools

import jax
import jax.numpy as jnp
from jax import lax
from jax.experimental import pallas as pl
from jax.experimental.pallas import tpu as pltpu
from jax.experimental.pallas import tpu_sc as plsc

NV = 10000
NE = 5000
NEP = 5008
NNZ = 320000
CHUNK = 6400
NGRP = CHUNK // 16
NCHUNK = NNZ // CHUNK
NC = 2
NS = 16
NW = NC * NS


def _mm_body(w_ref, x_ref, b_ref, o_ref, *, dims):
    o_ref[...] = lax.dot_general(
        w_ref[...], x_ref[...], dims, preferred_element_type=jnp.float32
    ) + b_ref[...]


def _mm_xt(W, X, b):
    do = W.shape[1]
    return pl.pallas_call(
        functools.partial(_mm_body, dims=(((0,), (1,)), ((), ()))),
        out_shape=jax.ShapeDtypeStruct((do, X.shape[0]), jnp.float32),
    )(W, X, b.reshape(do, 1))


def _mm_tt(W, Zt, b):
    do = W.shape[1]
    return pl.pallas_call(
        functools.partial(_mm_body, dims=(((0,), (0,)), ((), ()))),
        out_shape=jax.ShapeDtypeStruct((do, Zt.shape[1]), jnp.float32),
    )(W, Zt, b.reshape(do, 1))


def _pack_ids_body(v_ref, e_ref, o_ref):
    o_ref[...] = jnp.bitwise_or(jnp.left_shift(e_ref[...], 14), v_ref[...])


def _pack_ids(v32, e32):
    return pl.pallas_call(
        _pack_ids_body,
        out_shape=jax.ShapeDtypeStruct(v32.shape, jnp.int32),
    )(v32, e32)


def _zero_fill(ref, n16):
    zeros16 = jnp.zeros((16,), jnp.float32)

    def body(j, _):
        ref[pl.ds(j * 16, 16)] = zeros16
        return 0

    lax.fori_loop(0, n16, body, 0, unroll=4)


def _sc_body(*refs, C, first, relu):
    if first:
        yt, pids, zt, rvd_out, red_out = refs[:5]
        rest = refs[5:]
    else:
        yt, pids, rvd_in, red_in, zt = refs[:5]
        rest = refs[5:]
    npair = C // 2
    ab = rest[:C]
    eb = rest[C:2 * C]
    pxb = rest[2 * C:2 * C + npair]
    peb = rest[2 * C + npair:2 * C + 2 * npair]
    vd, ed, pk0, pk1, sem0, sem1 = rest[2 * C + 2 * npair:]

    wid = lax.axis_index("s") * NC + lax.axis_index("c")
    row0 = wid * C

    ones16 = jnp.full((16,), 1.0, jnp.float32)

    for c in range(C):
        pltpu.sync_copy(yt.at[pl.ds((row0 + c) * NV, NV)], ab[c])

    def pkx(j, _):
        s = pl.ds(j * 16, 16)
        for p in range(npair):
            w = plsc.pack(ab[2 * p][s], ab[2 * p + 1][s],
                          format=plsc.PackFormat.INTERLEAVED)
            pxb[p][s] = plsc.bitcast(w, jnp.float32)
        return 0
    lax.fori_loop(0, NV // 16, pkx, 0, unroll=4)

    for c in range(C):
        _zero_fill(eb[c], NEP // 16)
    if first:
        _zero_fill(vd, NV // 16)
        _zero_fill(ed, NEP // 16)
    else:
        pltpu.sync_copy(rvd_in, vd)
        pltpu.sync_copy(red_in, ed)

    def _issue(k, pkb, sem):
        base = pl.multiple_of(jnp.minimum(k * CHUNK, NNZ - CHUNK), 8)
        pltpu.async_copy(pids.at[pl.ds(base, CHUNK)], pkb, sem)

    def _drain(pkb, sem):
        pltpu.make_async_copy(pids.at[pl.ds(0, CHUNK)], pkb, sem).wait()

    def _stream(proc_chunk):
        _issue(0, pk0, sem0)
        _issue(1, pk1, sem1)

        def pair(kk, _):
            k0 = 2 * kk
            _drain(pk0, sem0)
            proc_chunk(pk0)
            _issue(k0 + 2, pk0, sem0)
            _drain(pk1, sem1)
            proc_chunk(pk1)
            _issue(k0 + 3, pk1, sem1)
            return 0

        lax.fori_loop(0, NCHUNK // 2, pair, 0)
        _drain(pk0, sem0)
        _drain(pk1, sem1)

    def _decode(pk16):
        v16 = jnp.bitwise_and(pk16, 16383)
        e16 = lax.shift_right_logical(pk16, 14)
        return v16, e16

    def pass1_chunk(pk):
        v0, e0 = _decode(pk[pl.ds(0, 16)])
        w0 = [plsc.load_gather(pxb[p], [v0]) for p in range(npair)]

        def grp(g, carry):
            v16, e16 = carry[0], carry[1]
            ws = carry[2:]
            offn = pl.ds(jnp.minimum((g + 1) * 16, CHUNK - 16), 16)
            vn, en = _decode(pk[offn])
            wn = [plsc.load_gather(pxb[p], [vn]) for p in range(npair)]
            if first:
                plsc.addupdate_scatter(vd, [v16], ones16)
                plsc.addupdate_scatter(ed, [e16], ones16)
            for p in range(npair):
                a, b = plsc.unpack(plsc.bitcast(ws[p], jnp.bfloat16),
                                   format=plsc.PackFormat.INTERLEAVED)
                plsc.addupdate_scatter(eb[2 * p], [e16], a)
                plsc.addupdate_scatter(eb[2 * p + 1], [e16], b)
            return (vn, en, *wn)

        lax.fori_loop(0, NGRP, grp, (v0, e0, *w0), unroll=8)

    _stream(pass1_chunk)

    if first:
        def rvd_loop(j, _):
            s = pl.ds(j * 16, 16)
            vd[s] = 1.0 / jnp.maximum(vd[s], 1.0)
            return 0
        lax.fori_loop(0, NV // 16, rvd_loop, 0, unroll=4)

        def red_loop(j, _):
            s = pl.ds(j * 16, 16)
            ed[s] = 1.0 / jnp.maximum(ed[s], 1.0)
            return 0
        lax.fori_loop(0, NEP // 16, red_loop, 0, unroll=4)

    def esc(j, _):
        s = pl.ds(j * 16, 16)
        r = ed[s]
        for p in range(npair):
            a = eb[2 * p][s] * r
            b = eb[2 * p + 1][s] * r
            w = plsc.pack(a, b, format=plsc.PackFormat.INTERLEAVED)
            peb[p][s] = plsc.bitcast(w, jnp.float32)
        return 0
    lax.fori_loop(0, NEP // 16, esc, 0, unroll=4)

    for c in range(C):
        _zero_fill(ab[c], NV // 16)

    def pass2_chunk(pk):
        v0, e0 = _decode(pk[pl.ds(0, 16)])
        w0 = [plsc.load_gather(peb[p], [e0]) for p in range(npair)]

        def grp(g, carry):
            v16, e16 = carry[0], carry[1]
            ws = carry[2:]
            offn = pl.ds(jnp.minimum((g + 1) * 16, CHUNK - 16), 16)
            vn, en = _decode(pk[offn])
            wn = [plsc.load_gather(peb[p], [en]) for p in range(npair)]
            for p in range(npair):
                a, b = plsc.unpack(plsc.bitcast(ws[p], jnp.bfloat16),
                                   format=plsc.PackFormat.INTERLEAVED)
                plsc.addupdate_scatter(ab[2 * p], [v16], a)
                plsc.addupdate_scatter(ab[2 * p + 1], [v16], b)
            return (vn, en, *wn)

        lax.fori_loop(0, NGRP, grp, (v0, e0, *w0), unroll=8)

    _stream(pass2_chunk)

    def vsc(j, _):
        s = pl.ds(j * 16, 16)
        r = vd[s]
        for c in range(C):
            x = ab[c][s] * r
            if relu:
                x = jnp.maximum(x, 0.0)
            ab[c][s] = x
        return 0
    lax.fori_loop(0, NV // 16, vsc, 0, unroll=4)

    for c in range(C):
        pltpu.sync_copy(ab[c], zt.at[pl.ds((row0 + c) * NV, NV)])

    if first:
        @pl.when(wid == 0)
        def _():
            pltpu.sync_copy(vd, rvd_out)
            pltpu.sync_copy(ed, red_out)


def _make_sc(C, first, relu):
    d = C * NW
    out_type = [jax.ShapeDtypeStruct((d * NV,), jnp.float32)]
    if first:
        out_type += [jax.ShapeDtypeStruct((NV,), jnp.float32),
                     jax.ShapeDtypeStruct((NEP,), jnp.float32)]
    scratch = (
        [pltpu.VMEM((NV,), jnp.float32) for _ in range(C)]
        + [pltpu.VMEM((NEP,), jnp.float32) for _ in range(C)]
        + [pltpu.VMEM((NV,), jnp.float32) for _ in range(C // 2)]
        + [pltpu.VMEM((NEP,), jnp.float32) for _ in range(C // 2)]
        + [
            pltpu.VMEM((NV,), jnp.float32),
            pltpu.VMEM((NEP,), jnp.float32),
            pltpu.VMEM((CHUNK,), jnp.int32),
            pltpu.VMEM((CHUNK,), jnp.int32),
            pltpu.SemaphoreType.DMA,
            pltpu.SemaphoreType.DMA,
        ]
    )
    mesh = plsc.VectorSubcoreMesh(core_axis_name="c", subcore_axis_name="s")
    return pl.kernel(
        functools.partial(_sc_body, C=C, first=first, relu=relu),
        out_type=out_type,
        mesh=mesh,
        scratch_types=scratch,
        compiler_params=pltpu.CompilerParams(needs_layout_passes=False),
    )


def kernel(X, v_ids, e_ids, W0, b0, W1, b1, W2, b2):
    pk = _pack_ids(v_ids.astype(jnp.int32), e_ids.astype(jnp.int32))

    sc_first = _make_sc(4, True, True)
    sc_mid = _make_sc(4, False, True)
    sc_last = _make_sc(2, False, False)

    y0 = _mm_xt(W0, X, b0)
    z0f, rvd, red = sc_first(y0.reshape(-1), pk)
    z0 = z0f.reshape(128, NV)
    y1 = _mm_tt(W1, z0, b1)
    (z1f,) = sc_mid(y1.reshape(-1), pk, rvd, red)
    z1 = z1f.reshape(128, NV)
    y2 = _mm_tt(W2, z1, b2)
    (z2f,) = sc_last(y2.reshape(-1), pk, rvd, red)
    return z2f.reshape(64, NV).T


# speedup vs baseline: 1.0419x; 1.0388x over previous
import functools

import jax
import jax.numpy as jnp
from jax import lax
from jax.experimental import pallas as pl
from jax.experimental.pallas import tpu as pltpu
from jax.experimental.pallas import tpu_sc as plsc

NV = 10000
NE = 5000
NEP = 5008
NNZ = 320000
CHUNK = 6400
NGRP = CHUNK // 16
NCHUNK = NNZ // CHUNK
NC = 2
NS = 16
NW = NC * NS


def _mm_body(w_ref, x_ref, b_ref, o_ref, *, dims):
    o_ref[...] = lax.dot_general(
        w_ref[...], x_ref[...], dims, preferred_element_type=jnp.float32
    ) + b_ref[...]


def _mm_xt(W, X, b):
    do = W.shape[1]
    return pl.pallas_call(
        functools.partial(_mm_body, dims=(((0,), (1,)), ((), ()))),
        out_shape=jax.ShapeDtypeStruct((do, X.shape[0]), jnp.float32),
    )(W, X, b.reshape(do, 1))


def _mm_tt(W, Zt, b):
    do = W.shape[1]
    return pl.pallas_call(
        functools.partial(_mm_body, dims=(((0,), (0,)), ((), ()))),
        out_shape=jax.ShapeDtypeStruct((do, Zt.shape[1]), jnp.float32),
    )(W, Zt, b.reshape(do, 1))


def _pack_ids_body(v_ref, e_ref, o_ref):
    o_ref[...] = jnp.bitwise_or(jnp.left_shift(e_ref[...], 14), v_ref[...])


def _pack_ids(v32, e32):
    return pl.pallas_call(
        _pack_ids_body,
        out_shape=jax.ShapeDtypeStruct(v32.shape, jnp.int32),
    )(v32, e32)


def _deg_body(pids, vparts, eparts, vd, ed, pkb):
    wid = lax.axis_index("s") * NC + lax.axis_index("c")
    npair_t = NNZ // NW
    _zero_fill(vd, NV // 16)
    _zero_fill(ed, NEP // 16)
    pltpu.sync_copy(pids.at[pl.ds(wid * npair_t, npair_t)], pkb)
    ones16 = jnp.full((16,), 1.0, jnp.float32)

    def grp(g, _):
        pk16 = pkb[pl.ds(g * 16, 16)]
        v16 = jnp.bitwise_and(pk16, 16383)
        e16 = lax.shift_right_logical(pk16, 14)
        plsc.addupdate_scatter(vd, [v16], ones16)
        plsc.addupdate_scatter(ed, [e16], ones16)
        return 0

    lax.fori_loop(0, npair_t // 16, grp, 0, unroll=8)
    pltpu.sync_copy(vd, vparts.at[pl.ds(wid * NV, NV)])
    pltpu.sync_copy(ed, eparts.at[pl.ds(wid * NEP, NEP)])


def _make_deg():
    out_type = [jax.ShapeDtypeStruct((NW * NV,), jnp.float32),
                jax.ShapeDtypeStruct((NW * NEP,), jnp.float32)]
    scratch = [pltpu.VMEM((NV,), jnp.float32),
               pltpu.VMEM((NEP,), jnp.float32),
               pltpu.VMEM((NNZ // NW,), jnp.int32)]
    mesh = plsc.VectorSubcoreMesh(core_axis_name="c", subcore_axis_name="s")
    return pl.kernel(
        _deg_body,
        out_type=out_type,
        mesh=mesh,
        scratch_types=scratch,
        compiler_params=pltpu.CompilerParams(needs_layout_passes=False),
    )


def _deg_reduce_body(vp_ref, ep_ref, rvd_ref, red_ref):
    rvd_ref[...] = 1.0 / jnp.maximum(jnp.sum(vp_ref[...], axis=0,
                                             keepdims=True), 1.0)
    red_ref[...] = 1.0 / jnp.maximum(jnp.sum(ep_ref[...], axis=0,
                                             keepdims=True), 1.0)


def _deg_reduce(vparts, eparts):
    rvd, red = pl.pallas_call(
        _deg_reduce_body,
        out_shape=[jax.ShapeDtypeStruct((1, NV), jnp.float32),
                   jax.ShapeDtypeStruct((1, NEP), jnp.float32)],
    )(vparts.reshape(NW, NV), eparts.reshape(NW, NEP))
    return rvd.reshape(-1), red.reshape(-1)


def _zero_fill(ref, n16):
    zeros16 = jnp.zeros((16,), jnp.float32)

    def body(j, _):
        ref[pl.ds(j * 16, 16)] = zeros16
        return 0

    lax.fori_loop(0, n16, body, 0, unroll=4)


def _sc_body(*refs, C, relu):
    yt, pids, rvd_in, red_in, zt = refs[:5]
    rest = refs[5:]
    npair = C // 2
    ab = rest[:C]
    eb = rest[C:2 * C]
    pxb = rest[2 * C:2 * C + npair]
    peb = rest[2 * C + npair:2 * C + 2 * npair]
    vd, ed, pk0, pk1, sem0, sem1 = rest[2 * C + 2 * npair:]

    wid = lax.axis_index("s") * NC + lax.axis_index("c")
    row0 = wid * C

    for c in range(C):
        pltpu.sync_copy(yt.at[pl.ds((row0 + c) * NV, NV)], ab[c])

    def pkx(j, _):
        s = pl.ds(j * 16, 16)
        for p in range(npair):
            w = plsc.pack(ab[2 * p][s], ab[2 * p + 1][s],
                          format=plsc.PackFormat.INTERLEAVED)
            pxb[p][s] = plsc.bitcast(w, jnp.float32)
        return 0
    lax.fori_loop(0, NV // 16, pkx, 0, unroll=4)

    for c in range(C):
        _zero_fill(eb[c], NEP // 16)
    pltpu.sync_copy(rvd_in, vd)
    pltpu.sync_copy(red_in, ed)

    def _issue(k, pkb, sem):
        base = pl.multiple_of(jnp.minimum(k * CHUNK, NNZ - CHUNK), 8)
        pltpu.async_copy(pids.at[pl.ds(base, CHUNK)], pkb, sem)

    def _drain(pkb, sem):
        pltpu.make_async_copy(pids.at[pl.ds(0, CHUNK)], pkb, sem).wait()

    def _stream(proc_chunk):
        _issue(0, pk0, sem0)
        _issue(1, pk1, sem1)

        def pair(kk, _):
            k0 = 2 * kk
            _drain(pk0, sem0)
            proc_chunk(pk0)
            _issue(k0 + 2, pk0, sem0)
            _drain(pk1, sem1)
            proc_chunk(pk1)
            _issue(k0 + 3, pk1, sem1)
            return 0

        lax.fori_loop(0, NCHUNK // 2, pair, 0)
        _drain(pk0, sem0)
        _drain(pk1, sem1)

    def _decode(pk16):
        v16 = jnp.bitwise_and(pk16, 16383)
        e16 = lax.shift_right_logical(pk16, 14)
        return v16, e16

    def pass1_chunk(pk):
        v0, e0 = _decode(pk[pl.ds(0, 16)])
        w0 = [plsc.load_gather(pxb[p], [v0]) for p in range(npair)]

        def grp(g, carry):
            v16, e16 = carry[0], carry[1]
            ws = carry[2:]
            offn = pl.ds(jnp.minimum((g + 1) * 16, CHUNK - 16), 16)
            vn, en = _decode(pk[offn])
            wn = [plsc.load_gather(pxb[p], [vn]) for p in range(npair)]
            for p in range(npair):
                a, b = plsc.unpack(plsc.bitcast(ws[p], jnp.bfloat16),
                                   format=plsc.PackFormat.INTERLEAVED)
                plsc.addupdate_scatter(eb[2 * p], [e16], a)
                plsc.addupdate_scatter(eb[2 * p + 1], [e16], b)
            return (vn, en, *wn)

        lax.fori_loop(0, NGRP, grp, (v0, e0, *w0), unroll=8)

    _stream(pass1_chunk)

    def esc(j, _):
        s = pl.ds(j * 16, 16)
        r = ed[s]
        for p in range(npair):
            a = eb[2 * p][s] * r
            b = eb[2 * p + 1][s] * r
            w = plsc.pack(a, b, format=plsc.PackFormat.INTERLEAVED)
            peb[p][s] = plsc.bitcast(w, jnp.float32)
        return 0
    lax.fori_loop(0, NEP // 16, esc, 0, unroll=4)

    for c in range(C):
        _zero_fill(ab[c], NV // 16)

    def pass2_chunk(pk):
        v0, e0 = _decode(pk[pl.ds(0, 16)])
        w0 = [plsc.load_gather(peb[p], [e0]) for p in range(npair)]

        def grp(g, carry):
            v16, e16 = carry[0], carry[1]
            ws = carry[2:]
            offn = pl.ds(jnp.minimum((g + 1) * 16, CHUNK - 16), 16)
            vn, en = _decode(pk[offn])
            wn = [plsc.load_gather(peb[p], [en]) for p in range(npair)]
            for p in range(npair):
                a, b = plsc.unpack(plsc.bitcast(ws[p], jnp.bfloat16),
                                   format=plsc.PackFormat.INTERLEAVED)
                plsc.addupdate_scatter(ab[2 * p], [v16], a)
                plsc.addupdate_scatter(ab[2 * p + 1], [v16], b)
            return (vn, en, *wn)

        lax.fori_loop(0, NGRP, grp, (v0, e0, *w0), unroll=8)

    _stream(pass2_chunk)

    def vsc(j, _):
        s = pl.ds(j * 16, 16)
        r = vd[s]
        for c in range(C):
            x = ab[c][s] * r
            if relu:
                x = jnp.maximum(x, 0.0)
            ab[c][s] = x
        return 0
    lax.fori_loop(0, NV // 16, vsc, 0, unroll=4)

    for c in range(C):
        pltpu.sync_copy(ab[c], zt.at[pl.ds((row0 + c) * NV, NV)])


def _make_sc(C, relu):
    d = C * NW
    out_type = [jax.ShapeDtypeStruct((d * NV,), jnp.float32)]
    scratch = (
        [pltpu.VMEM((NV,), jnp.float32) for _ in range(C)]
        + [pltpu.VMEM((NEP,), jnp.float32) for _ in range(C)]
        + [pltpu.VMEM((NV,), jnp.float32) for _ in range(C // 2)]
        + [pltpu.VMEM((NEP,), jnp.float32) for _ in range(C // 2)]
        + [
            pltpu.VMEM((NV,), jnp.float32),
            pltpu.VMEM((NEP,), jnp.float32),
            pltpu.VMEM((CHUNK,), jnp.int32),
            pltpu.VMEM((CHUNK,), jnp.int32),
            pltpu.SemaphoreType.DMA,
            pltpu.SemaphoreType.DMA,
        ]
    )
    mesh = plsc.VectorSubcoreMesh(core_axis_name="c", subcore_axis_name="s")
    return pl.kernel(
        functools.partial(_sc_body, C=C, relu=relu),
        out_type=out_type,
        mesh=mesh,
        scratch_types=scratch,
        compiler_params=pltpu.CompilerParams(needs_layout_passes=False),
    )


def kernel(X, v_ids, e_ids, W0, b0, W1, b1, W2, b2):
    pk = _pack_ids(v_ids.astype(jnp.int32), e_ids.astype(jnp.int32))
    vparts, eparts = _make_deg()(pk)
    rvd, red = _deg_reduce(vparts, eparts)

    sc_mid = _make_sc(4, True)
    sc_last = _make_sc(2, False)

    y0 = _mm_xt(W0, X, b0)
    (z0f,) = sc_mid(y0.reshape(-1), pk, rvd, red)
    z0 = z0f.reshape(128, NV)
    y1 = _mm_tt(W1, z0, b1)
    (z1f,) = sc_mid(y1.reshape(-1), pk, rvd, red)
    z1 = z1f.reshape(128, NV)
    y2 = _mm_tt(W2, z1, b2)
    (z2f,) = sc_last(y2.reshape(-1), pk, rvd, red)
    return z2f.reshape(64, NV).T


# speedup vs baseline: 1.0557x; 1.0132x over previous
import functools

import jax
import jax.numpy as jnp
from jax import lax
from jax.experimental import pallas as pl
from jax.experimental.pallas import tpu as pltpu
from jax.experimental.pallas import tpu_sc as plsc

NV = 10000
NE = 5000
NEP = 5008
NNZ = 320000
CHUNK = 6400
NGRP = CHUNK // 16
NCHUNK = NNZ // CHUNK
NC = 2
NS = 16
NW = NC * NS


def _mm_body(we_ref, wo_ref, x_ref, be_ref, bo_ref, o_ref, *, dims):
    ye = lax.dot_general(
        we_ref[...], x_ref[...], dims, preferred_element_type=jnp.float32
    ) + be_ref[...]
    yo = lax.dot_general(
        wo_ref[...], x_ref[...], dims, preferred_element_type=jnp.float32
    ) + bo_ref[...]
    lo = lax.bitcast_convert_type(
        ye.astype(jnp.bfloat16), jnp.uint16).astype(jnp.uint32)
    hi = lax.bitcast_convert_type(
        yo.astype(jnp.bfloat16), jnp.uint16).astype(jnp.uint32)
    o_ref[...] = lax.bitcast_convert_type(
        jnp.bitwise_or(lo, jnp.left_shift(hi, 16)), jnp.int32)


def _mm_pk(W, Xt, b, x_contract):
    do = W.shape[1]
    nv = Xt.shape[1 - x_contract]
    return pl.pallas_call(
        functools.partial(_mm_body, dims=(((0,), (x_contract,)), ((), ()))),
        out_shape=jax.ShapeDtypeStruct((do // 2, nv), jnp.int32),
    )(W[:, 0::2], W[:, 1::2], Xt,
      b[0::2].reshape(do // 2, 1), b[1::2].reshape(do // 2, 1))


def _pack_ids_body(v_ref, e_ref, o_ref):
    o_ref[...] = jnp.bitwise_or(jnp.left_shift(e_ref[...], 14), v_ref[...])


def _pack_ids(v32, e32):
    return pl.pallas_call(
        _pack_ids_body,
        out_shape=jax.ShapeDtypeStruct(v32.shape, jnp.int32),
    )(v32, e32)


def _deg_body(pids, vparts, eparts, vd, ed, pkb):
    wid = lax.axis_index("s") * NC + lax.axis_index("c")
    npair_t = NNZ // NW
    _zero_fill(vd, NV // 16)
    _zero_fill(ed, NEP // 16)
    pltpu.sync_copy(pids.at[pl.ds(wid * npair_t, npair_t)], pkb)
    ones16 = jnp.full((16,), 1.0, jnp.float32)

    def grp(g, _):
        pk16 = pkb[pl.ds(g * 16, 16)]
        v16 = jnp.bitwise_and(pk16, 16383)
        e16 = lax.shift_right_logical(pk16, 14)
        plsc.addupdate_scatter(vd, [v16], ones16)
        plsc.addupdate_scatter(ed, [e16], ones16)
        return 0

    lax.fori_loop(0, npair_t // 16, grp, 0, unroll=8)
    pltpu.sync_copy(vd, vparts.at[pl.ds(wid * NV, NV)])
    pltpu.sync_copy(ed, eparts.at[pl.ds(wid * NEP, NEP)])


def _make_deg():
    out_type = [jax.ShapeDtypeStruct((NW * NV,), jnp.float32),
                jax.ShapeDtypeStruct((NW * NEP,), jnp.float32)]
    scratch = [pltpu.VMEM((NV,), jnp.float32),
               pltpu.VMEM((NEP,), jnp.float32),
               pltpu.VMEM((NNZ // NW,), jnp.int32)]
    mesh = plsc.VectorSubcoreMesh(core_axis_name="c", subcore_axis_name="s")
    return pl.kernel(
        _deg_body,
        out_type=out_type,
        mesh=mesh,
        scratch_types=scratch,
        compiler_params=pltpu.CompilerParams(needs_layout_passes=False),
    )


def _deg_reduce_body(vp_ref, ep_ref, rvd_ref, red_ref):
    rvd_ref[...] = 1.0 / jnp.maximum(jnp.sum(vp_ref[...], axis=0,
                                             keepdims=True), 1.0)
    red_ref[...] = 1.0 / jnp.maximum(jnp.sum(ep_ref[...], axis=0,
                                             keepdims=True), 1.0)


def _deg_reduce(vparts, eparts):
    rvd, red = pl.pallas_call(
        _deg_reduce_body,
        out_shape=[jax.ShapeDtypeStruct((1, NV), jnp.float32),
                   jax.ShapeDtypeStruct((1, NEP), jnp.float32)],
    )(vparts.reshape(NW, NV), eparts.reshape(NW, NEP))
    return rvd.reshape(-1), red.reshape(-1)


def _zero_fill(ref, n16):
    zeros16 = jnp.zeros((16,), jnp.float32)

    def body(j, _):
        ref[pl.ds(j * 16, 16)] = zeros16
        return 0

    lax.fori_loop(0, n16, body, 0, unroll=4)


def _sc_body(*refs, C, relu):
    yt, pids, rvd_in, red_in, zt = refs[:5]
    rest = refs[5:]
    npair = C // 2
    ab = rest[:C]
    eb = rest[C:2 * C]
    pxb = rest[2 * C:2 * C + npair]
    peb = rest[2 * C + npair:2 * C + 2 * npair]
    vd, ed, pk0, pk1, sem0, sem1 = rest[2 * C + 2 * npair:]

    wid = lax.axis_index("s") * NC + lax.axis_index("c")
    row0 = wid * C

    for p in range(npair):
        pltpu.sync_copy(yt.at[pl.ds((wid * npair + p) * NV, NV)], pxb[p])

    for c in range(C):
        _zero_fill(eb[c], NEP // 16)
    pltpu.sync_copy(rvd_in, vd)
    pltpu.sync_copy(red_in, ed)

    def _issue(k, pkb, sem):
        base = pl.multiple_of(jnp.minimum(k * CHUNK, NNZ - CHUNK), 8)
        pltpu.async_copy(pids.at[pl.ds(base, CHUNK)], pkb, sem)

    def _drain(pkb, sem):
        pltpu.make_async_copy(pids.at[pl.ds(0, CHUNK)], pkb, sem).wait()

    def _stream(proc_chunk):
        _issue(0, pk0, sem0)
        _issue(1, pk1, sem1)

        def pair(kk, _):
            k0 = 2 * kk
            _drain(pk0, sem0)
            proc_chunk(pk0)
            _issue(k0 + 2, pk0, sem0)
            _drain(pk1, sem1)
            proc_chunk(pk1)
            _issue(k0 + 3, pk1, sem1)
            return 0

        lax.fori_loop(0, NCHUNK // 2, pair, 0)
        _drain(pk0, sem0)
        _drain(pk1, sem1)

    def _decode(pk16):
        v16 = jnp.bitwise_and(pk16, 16383)
        e16 = lax.shift_right_logical(pk16, 14)
        return v16, e16

    def pass1_chunk(pk):
        v0, e0 = _decode(pk[pl.ds(0, 16)])
        w0 = [plsc.load_gather(pxb[p], [v0]) for p in range(npair)]

        def grp(g, carry):
            v16, e16 = carry[0], carry[1]
            ws = carry[2:]
            offn = pl.ds(jnp.minimum((g + 1) * 16, CHUNK - 16), 16)
            vn, en = _decode(pk[offn])
            wn = [plsc.load_gather(pxb[p], [vn]) for p in range(npair)]
            for p in range(npair):
                a, b = plsc.unpack(plsc.bitcast(ws[p], jnp.bfloat16),
                                   format=plsc.PackFormat.INTERLEAVED)
                plsc.addupdate_scatter(eb[2 * p], [e16], a)
                plsc.addupdate_scatter(eb[2 * p + 1], [e16], b)
            return (vn, en, *wn)

        lax.fori_loop(0, NGRP, grp, (v0, e0, *w0), unroll=8)

    _stream(pass1_chunk)

    def esc(j, _):
        s = pl.ds(j * 16, 16)
        r = ed[s]
        for p in range(npair):
            a = eb[2 * p][s] * r
            b = eb[2 * p + 1][s] * r
            w = plsc.pack(a, b, format=plsc.PackFormat.INTERLEAVED)
            peb[p][s] = plsc.bitcast(w, jnp.float32)
        return 0
    lax.fori_loop(0, NEP // 16, esc, 0, unroll=4)

    for c in range(C):
        _zero_fill(ab[c], NV // 16)

    def pass2_chunk(pk):
        v0, e0 = _decode(pk[pl.ds(0, 16)])
        w0 = [plsc.load_gather(peb[p], [e0]) for p in range(npair)]

        def grp(g, carry):
            v16, e16 = carry[0], carry[1]
            ws = carry[2:]
            offn = pl.ds(jnp.minimum((g + 1) * 16, CHUNK - 16), 16)
            vn, en = _decode(pk[offn])
            wn = [plsc.load_gather(peb[p], [en]) for p in range(npair)]
            for p in range(npair):
                a, b = plsc.unpack(plsc.bitcast(ws[p], jnp.bfloat16),
                                   format=plsc.PackFormat.INTERLEAVED)
                plsc.addupdate_scatter(ab[2 * p], [v16], a)
                plsc.addupdate_scatter(ab[2 * p + 1], [v16], b)
            return (vn, en, *wn)

        lax.fori_loop(0, NGRP, grp, (v0, e0, *w0), unroll=8)

    _stream(pass2_chunk)

    def vsc(j, _):
        s = pl.ds(j * 16, 16)
        r = vd[s]
        for c in range(C):
            x = ab[c][s] * r
            if relu:
                x = jnp.maximum(x, 0.0)
            ab[c][s] = x
        return 0
    lax.fori_loop(0, NV // 16, vsc, 0, unroll=4)

    for c in range(C):
        pltpu.sync_copy(ab[c], zt.at[pl.ds((row0 + c) * NV, NV)])


def _make_sc(C, relu):
    d = C * NW
    out_type = [jax.ShapeDtypeStruct((d * NV,), jnp.float32)]
    scratch = (
        [pltpu.VMEM((NV,), jnp.float32) for _ in range(C)]
        + [pltpu.VMEM((NEP,), jnp.float32) for _ in range(C)]
        + [pltpu.VMEM((NV,), jnp.int32) for _ in range(C // 2)]
        + [pltpu.VMEM((NEP,), jnp.float32) for _ in range(C // 2)]
        + [
            pltpu.VMEM((NV,), jnp.float32),
            pltpu.VMEM((NEP,), jnp.float32),
            pltpu.VMEM((CHUNK,), jnp.int32),
            pltpu.VMEM((CHUNK,), jnp.int32),
            pltpu.SemaphoreType.DMA,
            pltpu.SemaphoreType.DMA,
        ]
    )
    mesh = plsc.VectorSubcoreMesh(core_axis_name="c", subcore_axis_name="s")
    return pl.kernel(
        functools.partial(_sc_body, C=C, relu=relu),
        out_type=out_type,
        mesh=mesh,
        scratch_types=scratch,
        compiler_params=pltpu.CompilerParams(needs_layout_passes=False),
    )


def kernel(X, v_ids, e_ids, W0, b0, W1, b1, W2, b2):
    pk = _pack_ids(v_ids.astype(jnp.int32), e_ids.astype(jnp.int32))
    vparts, eparts = _make_deg()(pk)
    rvd, red = _deg_reduce(vparts, eparts)

    sc_mid = _make_sc(4, True)
    sc_last = _make_sc(2, False)

    y0 = _mm_pk(W0, X, b0, 1)
    (z0f,) = sc_mid(y0.reshape(-1), pk, rvd, red)
    z0 = z0f.reshape(128, NV)
    y1 = _mm_pk(W1, z0, b1, 0)
    (z1f,) = sc_mid(y1.reshape(-1), pk, rvd, red)
    z1 = z1f.reshape(128, NV)
    y2 = _mm_pk(W2, z1, b2, 0)
    (z2f,) = sc_last(y2.reshape(-1), pk, rvd, red)
    return z2f.reshape(64, NV).T
